# baseline (speedup 1.0000x reference)
import jax
import jax.numpy as jnp
from jax.experimental import pallas as pl
from jax.experimental.pallas import tpu as pltpu

OH = 7
OW = 7
WIN = 80
DEPTH = 6
MROWS = 56
KDIM = 14 * WIN


def _roi_kernel(bidx_ref, cxa_ref, ysa_ref,
                feat_ref, wq_ref, sk_ref, rk_ref,
                out_ref,
                strips, sems):
    tpc = pl.num_programs(1)
    c = pl.program_id(0)
    t = pl.program_id(1)
    n = c * tpc + t

    def issue(m):
        slot = jax.lax.rem(m, DEPTH)
        b = bidx_ref[m]
        cxv = pl.multiple_of(cxa_ref[m], 8)
        for j in range(OH):
            ys = ysa_ref[m, j]
            pltpu.make_async_copy(
                feat_ref.at[b, pl.ds(ys, 2), pl.ds(cxv, WIN), :],
                strips.at[slot, pl.ds(2 * j, 2)],
                sems.at[slot, j],
            ).start()

    @pl.when(t == 0)
    def _():
        for d in range(DEPTH - 1):
            issue(n + d)

    @pl.when(t + DEPTH - 1 < tpc)
    def _():
        issue(n + DEPTH - 1)

    slot = jax.lax.rem(n, DEPTH)
    for j in range(OH):
        pltpu.make_async_copy(
            strips.at[slot, pl.ds(2 * j, 2)],
            strips.at[slot, pl.ds(2 * j, 2)],
            sems.at[slot, j],
        ).wait()

    wt = wq_ref[0]
    sk = sk_ref[...]
    rk = rk_ref[...]

    def col(i):
        return wt[:, i:i + 1]

    zero = jnp.float32(0.0)
    rowterm = (jnp.where(sk == col(0), col(2), zero)
               + jnp.where(sk == col(1), col(3), zero))
    colterm = (jnp.where(rk == col(4), col(6), zero)
               + jnp.where(rk == col(5), col(7), zero))
    w = rowterm * colterm

    s = strips[slot].reshape(KDIM, strips.shape[-1])
    res = jnp.dot(w, s, preferred_element_type=jnp.float32)
    out_ref[0] = res[:OH * OW]


def kernel(features, rois):
    B, C, H, W = features.shape
    N = rois.shape[0]
    f32 = jnp.float32

    b = rois[:, 0]
    coords = rois[:, 1:5].astype(f32) * 1.0
    x1, y1, x2, y2 = coords[:, 0], coords[:, 1], coords[:, 2], coords[:, 3]

    jy = jnp.arange(OH, dtype=f32)
    jx = jnp.arange(OW, dtype=f32)
    sy = y1[:, None] + jy[None, :] * (y2 - y1)[:, None] / (OH - 1)
    sx = x1[:, None] + jx[None, :] * (x2 - x1)[:, None] / (OW - 1)

    y0f = jnp.floor(sy)
    x0f = jnp.floor(sx)
    wy = sy - y0f
    wx = sx - x0f
    y0 = jnp.clip(y0f.astype(jnp.int32), 0, H - 1)
    y1i = jnp.clip(y0 + 1, 0, H - 1)
    x0 = jnp.clip(x0f.astype(jnp.int32), 0, W - 1)
    x1i = jnp.clip(x0 + 1, 0, W - 1)

    ys = jnp.minimum(y0, H - 2)
    rel_top = (y0 - ys).astype(f32)
    rel_bot = (y1i - ys).astype(f32)
    a0 = (1.0 - wy) * (rel_top == 0) + wy * (rel_bot == 0)
    a1 = (1.0 - wy) * (rel_top == 1) + wy * (rel_bot == 1)

    x1int = rois[:, 1]
    cx = jnp.minimum((x1int // 8) * 8, W - WIN)
    c0 = (x0 - cx[:, None]).astype(f32)
    c1 = (x1i - cx[:, None]).astype(f32)
    b0 = 1.0 - wx
    b1 = wx

    rr = jnp.arange(OH * OW)
    jv = rr // OW
    iv = rr % OW
    t0 = jnp.broadcast_to((2 * jv).astype(f32), (N, OH * OW))
    t1 = t0 + 1.0
    wq49 = jnp.stack(
        [t0, t1, a0[:, jv], a1[:, jv], c0[:, iv], c1[:, iv],
         b0[:, iv], b1[:, iv]], axis=-1)
    wq = jnp.concatenate(
        [wq49, jnp.zeros((N, MROWS - OH * OW, 8), f32)], axis=1)

    kk = jnp.arange(KDIM, dtype=jnp.int32)
    sk = (kk // WIN).astype(f32)[None, :]
    rk = (kk % WIN).astype(f32)[None, :]

    feat_nhwc = jnp.transpose(features, (0, 2, 3, 1))

    ysa = jnp.concatenate([ys, jnp.zeros((N, 1), jnp.int32)], axis=1)

    tpc = N // 2
    grid_spec = pltpu.PrefetchScalarGridSpec(
        num_scalar_prefetch=3,
        grid=(2, tpc),
        in_specs=[
            pl.BlockSpec(memory_space=pl.ANY),
            pl.BlockSpec((1, MROWS, 8), lambda c, t, *_: (c * tpc + t, 0, 0)),
            pl.BlockSpec((1, KDIM), lambda c, t, *_: (0, 0)),
            pl.BlockSpec((1, KDIM), lambda c, t, *_: (0, 0)),
        ],
        out_specs=pl.BlockSpec((1, OH * OW, C),
                               lambda c, t, *_: (c * tpc + t, 0, 0)),
        scratch_shapes=[
            pltpu.VMEM((DEPTH, 2 * OH, WIN, C), f32),
            pltpu.SemaphoreType.DMA((DEPTH, OH)),
        ],
    )
    out = pl.pallas_call(
        _roi_kernel,
        grid_spec=grid_spec,
        out_shape=jax.ShapeDtypeStruct((N, OH * OW, C), f32),
        compiler_params=pltpu.CompilerParams(
            dimension_semantics=("parallel", "arbitrary")),
    )(b, cx, ysa, feat_nhwc, wq, sk, rk)

    return out.transpose(0, 2, 1).reshape(N, C, OH, OW)
